# baseline (speedup 1.0000x reference)
import functools

import jax
import jax.numpy as jnp
from jax import lax
from jax.experimental import pallas as pl
from jax.experimental.pallas import tpu as pltpu
from jax.experimental.pallas import tpu_sc as plsc

BATCH = 1024
VOCAB = 100000
EMBED = 32


_VT = 1024


def _logits_body(ctx_ref, w_ref, b_ref, out_ref):
    acc = lax.dot_general(
        ctx_ref[...],
        w_ref[...],
        dimension_numbers=(((1,), (1,)), ((), ())),
        preferred_element_type=jnp.float32,
    )
    out_ref[...] = acc + b_ref[...]


def _logits(ctx, w, bias):
    return pl.pallas_call(
        _logits_body,
        grid=(pl.cdiv(VOCAB, _VT),),
        in_specs=[
            pl.BlockSpec((BATCH, EMBED), lambda j: (0, 0)),
            pl.BlockSpec((_VT, EMBED), lambda j: (j, 0)),
            pl.BlockSpec((1, _VT), lambda j: (0, j)),
        ],
        out_specs=pl.BlockSpec((BATCH, _VT), lambda j: (0, j)),
        out_shape=jax.ShapeDtypeStruct((BATCH, VOCAB), jnp.float32),
        compiler_params=pltpu.CompilerParams(
            dimension_semantics=("parallel",),
        ),
    )(ctx, w, bias.reshape(1, VOCAB))


try:
    _info = plsc.get_sparse_core_info()
    _NC, _NS = _info.num_cores, _info.num_subcores
except Exception:
    _NC, _NS = 2, 16
_NW = _NC * _NS
_BPW = BATCH // _NW

_sc_mesh = plsc.VectorSubcoreMesh(core_axis_name="c", subcore_axis_name="s")


@functools.partial(
    pl.kernel,
    mesh=_sc_mesh,
    out_type=jax.ShapeDtypeStruct((BATCH, EMBED), jnp.float32),
    scratch_types=[
        pltpu.VMEM((_BPW,), jnp.int32),
        pltpu.VMEM((_BPW, EMBED), jnp.float32),
        pltpu.SemaphoreType.DMA,
    ],
    compiler_params=pltpu.CompilerParams(use_tc_tiling_on_sc=False),
)
def _sc_gather(table_hbm, idx_hbm, out_hbm, idx_v, rows_v, sem):
    wid = lax.axis_index("s") * _NC + lax.axis_index("c")
    base = wid * _BPW
    pltpu.sync_copy(idx_hbm.at[pl.ds(base, _BPW)], idx_v)
    pltpu.async_copy(table_hbm.at[idx_v], rows_v, sem).wait()
    pltpu.sync_copy(rows_v, out_hbm.at[pl.ds(base, _BPW)])


def kernel(context_features, labels, label_embed_weight, out_fc_weight, out_fc_bias):
    out_logits = _logits(context_features, out_fc_weight, out_fc_bias)
    out_embeddings = _sc_gather(label_embed_weight, labels.astype(jnp.int32))
    return (context_features, out_logits, out_embeddings, label_embed_weight)


# speedup vs baseline: 1.0329x; 1.0329x over previous
import functools

import jax
import jax.numpy as jnp
from jax import lax
from jax.experimental import pallas as pl
from jax.experimental.pallas import tpu as pltpu
from jax.experimental.pallas import tpu_sc as plsc

BATCH = 1024
VOCAB = 100000
EMBED = 32


_VT = 4096


def _logits_body(ctx_ref, w_ref, b_ref, out_ref):
    acc = lax.dot_general(
        ctx_ref[...],
        w_ref[...],
        dimension_numbers=(((1,), (1,)), ((), ())),
        preferred_element_type=jnp.float32,
    )
    out_ref[...] = acc + b_ref[...]


def _logits(ctx, w, bias):
    return pl.pallas_call(
        _logits_body,
        grid=(pl.cdiv(VOCAB, _VT),),
        in_specs=[
            pl.BlockSpec((BATCH, EMBED), lambda j: (0, 0)),
            pl.BlockSpec((_VT, EMBED), lambda j: (j, 0)),
            pl.BlockSpec((1, _VT), lambda j: (0, j)),
        ],
        out_specs=pl.BlockSpec((BATCH, _VT), lambda j: (0, j)),
        out_shape=jax.ShapeDtypeStruct((BATCH, VOCAB), jnp.float32),
        compiler_params=pltpu.CompilerParams(
            dimension_semantics=("parallel",),
        ),
    )(ctx, w, bias.reshape(1, VOCAB))


try:
    _info = plsc.get_sparse_core_info()
    _NC, _NS = _info.num_cores, _info.num_subcores
except Exception:
    _NC, _NS = 2, 16
_NW = _NC * _NS
_BPW = BATCH // _NW

_sc_mesh = plsc.VectorSubcoreMesh(core_axis_name="c", subcore_axis_name="s")


@functools.partial(
    pl.kernel,
    mesh=_sc_mesh,
    out_type=jax.ShapeDtypeStruct((BATCH, EMBED), jnp.float32),
    scratch_types=[
        pltpu.VMEM((_BPW,), jnp.int32),
        pltpu.VMEM((_BPW, EMBED), jnp.float32),
        pltpu.SemaphoreType.DMA,
    ],
    compiler_params=pltpu.CompilerParams(use_tc_tiling_on_sc=False),
)
def _sc_gather(table_hbm, idx_hbm, out_hbm, idx_v, rows_v, sem):
    wid = lax.axis_index("s") * _NC + lax.axis_index("c")
    base = wid * _BPW
    pltpu.sync_copy(idx_hbm.at[pl.ds(base, _BPW)], idx_v)
    pltpu.async_copy(table_hbm.at[idx_v], rows_v, sem).wait()
    pltpu.sync_copy(rows_v, out_hbm.at[pl.ds(base, _BPW)])


def kernel(context_features, labels, label_embed_weight, out_fc_weight, out_fc_bias):
    out_logits = _logits(context_features, out_fc_weight, out_fc_bias)
    out_embeddings = _sc_gather(label_embed_weight, labels.astype(jnp.int32))
    return (context_features, out_logits, out_embeddings, label_embed_weight)


# speedup vs baseline: 1.0865x; 1.0519x over previous
import functools

import jax
import jax.numpy as jnp
from jax import lax
from jax.experimental import pallas as pl
from jax.experimental.pallas import tpu as pltpu
from jax.experimental.pallas import tpu_sc as plsc

BATCH = 1024
VOCAB = 100000
EMBED = 32


_VT = 4096


def _logits_body(ctx_ref, w_ref, b_ref, out_ref):
    acc = lax.dot_general(
        ctx_ref[...],
        w_ref[...],
        dimension_numbers=(((1,), (1,)), ((), ())),
        preferred_element_type=jnp.float32,
    )
    out_ref[...] = acc + b_ref[...]


def _logits(ctx, w, bias):
    return pl.pallas_call(
        _logits_body,
        grid=(pl.cdiv(VOCAB, _VT),),
        in_specs=[
            pl.BlockSpec((BATCH, EMBED), lambda j: (0, 0)),
            pl.BlockSpec((_VT, EMBED), lambda j: (j, 0)),
            pl.BlockSpec((1, _VT), lambda j: (0, j)),
        ],
        out_specs=pl.BlockSpec((BATCH, _VT), lambda j: (0, j)),
        out_shape=jax.ShapeDtypeStruct((BATCH, VOCAB), jnp.float32),
        compiler_params=pltpu.CompilerParams(
            dimension_semantics=("parallel",),
        ),
    )(ctx, w, bias.reshape(1, VOCAB))


try:
    _info = plsc.get_sparse_core_info()
    _NC, _NS = _info.num_cores, _info.num_subcores
except Exception:
    _NC, _NS = 2, 16
_NW = _NC * _NS
_BPW = BATCH // _NW

_sc_mesh = plsc.VectorSubcoreMesh(core_axis_name="c", subcore_axis_name="s")


@functools.partial(
    pl.kernel,
    mesh=_sc_mesh,
    out_type=jax.ShapeDtypeStruct((BATCH, EMBED), jnp.float32),
    scratch_types=[
        pltpu.VMEM((_BPW,), jnp.int32),
        pltpu.VMEM((_BPW, EMBED), jnp.float32),
        pltpu.SemaphoreType.DMA,
    ],
    compiler_params=pltpu.CompilerParams(use_tc_tiling_on_sc=False),
)
def _sc_gather(table_hbm, idx_hbm, out_hbm, idx_v, rows_v, sem):
    wid = lax.axis_index("s") * _NC + lax.axis_index("c")
    base = wid * _BPW
    pltpu.sync_copy(idx_hbm.at[pl.ds(base, _BPW)], idx_v)
    pltpu.async_copy(table_hbm.at[idx_v], rows_v, sem).wait()
    pltpu.sync_copy(rows_v, out_hbm.at[pl.ds(base, _BPW)])


def kernel(context_features, labels, label_embed_weight, out_fc_weight, out_fc_bias):
    out_logits = _logits(context_features, out_fc_weight, out_fc_bias)
    out_embeddings = jnp.take(label_embed_weight, labels, axis=0)
    return (context_features, out_logits, out_embeddings, label_embed_weight)
